# baseline (speedup 1.0000x reference)
import functools

import jax
import jax.numpy as jnp
from jax import lax
from jax.experimental import pallas as pl
from jax.experimental.pallas import tpu as pltpu
from jax.experimental.pallas import tpu_sc as plsc

_NC = 2
_NS = 16
_CH = 4


def kernel(x, pe_weight):
    batch = x.shape[0]
    max_len, d_model = pe_weight.shape
    nw = _NC * _NS
    rows_per_w = batch // nw
    n_copies = rows_per_w // _CH

    mesh = plsc.VectorSubcoreMesh(core_axis_name="c", subcore_axis_name="s")

    @functools.partial(
        pl.kernel,
        mesh=mesh,
        out_type=jax.ShapeDtypeStruct((batch, max_len, d_model), pe_weight.dtype),
        scratch_types=[
            pltpu.VMEM((_CH, max_len, d_model), pe_weight.dtype),
            pltpu.SemaphoreType.DMA,
        ],
    )
    def sc_bcast(pe_hbm, out_hbm, stage_v, sem):
        wid = lax.axis_index("s") * _NC + lax.axis_index("c")
        base = wid * rows_per_w
        stage_copies = [
            pltpu.make_async_copy(pe_hbm, stage_v.at[j], sem)
            for j in range(_CH)
        ]
        for c in stage_copies:
            c.start()
        for c in stage_copies:
            c.wait()
        copies = [
            pltpu.make_async_copy(
                stage_v, out_hbm.at[pl.ds(base + i * _CH, _CH)], sem
            )
            for i in range(n_copies)
        ]
        for c in copies:
            c.start()
        for c in copies:
            c.wait()

    return sc_bcast(pe_weight)


# speedup vs baseline: 1.6590x; 1.6590x over previous
import jax
import jax.numpy as jnp
from jax.experimental import pallas as pl

_BB = 16


def _bcast_body(pe_ref, out_ref):
    out_ref[...] = jnp.broadcast_to(pe_ref[...][None, :, :], out_ref.shape)


def kernel(x, pe_weight):
    batch = x.shape[0]
    max_len, d_model = pe_weight.shape
    bb = _BB if batch % _BB == 0 else 1
    return pl.pallas_call(
        _bcast_body,
        grid=(batch // bb,),
        in_specs=[pl.BlockSpec((max_len, d_model), lambda i: (0, 0))],
        out_specs=pl.BlockSpec((bb, max_len, d_model), lambda i: (i, 0, 0)),
        out_shape=jax.ShapeDtypeStruct((batch, max_len, d_model), pe_weight.dtype),
    )(pe_weight)


# speedup vs baseline: 2.1538x; 1.2983x over previous
import jax
import jax.numpy as jnp
from jax.experimental import pallas as pl
from jax.experimental.pallas import tpu as pltpu

_BB = 32


def _bcast_body(pe_ref, out_ref):
    out_ref[...] = jnp.broadcast_to(pe_ref[...][None, :, :], out_ref.shape)


def kernel(x, pe_weight):
    batch = x.shape[0]
    max_len, d_model = pe_weight.shape
    bb = _BB if batch % _BB == 0 else 1
    return pl.pallas_call(
        _bcast_body,
        grid=(batch // bb,),
        in_specs=[pl.BlockSpec(memory_space=pltpu.MemorySpace.VMEM)],
        out_specs=pl.BlockSpec((bb, max_len, d_model), lambda i: (i, 0, 0)),
        out_shape=jax.ShapeDtypeStruct((batch, max_len, d_model), pe_weight.dtype),
    )(pe_weight)
